# baseline (speedup 1.0000x reference)
import functools

import jax
import jax.numpy as jnp
from jax import lax
from jax.experimental import pallas as pl
from jax.experimental.pallas import tpu as pltpu
from jax.experimental.pallas import tpu_sc as plsc

B, Q, T, K, D = 16, 8, 2048, 1024, 128
NC, NS, L = 2, 16, 16
NW = NC * NS
C = 128
NCH = T // C
NIT = B * NCH // NW
STRIDE = 7
ZROW = Q * K
VPR = D // L
NZ = 64
NTAB = Q * K + NZ
RPS = (Q * K) // NS
ZPS = NZ // NS

_mesh = plsc.VectorSubcoreMesh(core_axis_name="c", subcore_axis_name="s")


@functools.partial(
    pl.kernel,
    out_type=jax.ShapeDtypeStruct((B, T, D), jnp.float32),
    mesh=_mesh,
    scratch_types=[
        pltpu.VMEM((NIT, Q, C), jnp.int32),
        pltpu.VMEM((NIT, Q, C), jnp.int32),
        pltpu.VMEM((2, C, D), jnp.float32),
        pltpu.VMEM((L,), jnp.int32),
        pltpu.VMEM_SHARED((NTAB, D), jnp.float32),
        pltpu.SemaphoreType.DMA,
        pltpu.SemaphoreType.DMA,
        pltpu.SemaphoreType.DMA,
        pltpu.SemaphoreType.DMA,
        pltpu.SemaphoreType.DMA,
        pltpu.SemaphoreType.DMA,
    ],
)
def _dequant(x_hbm, xlen_hbm, tab_hbm, out_hbm,
             xbuf, idxbuf, acc, lenbuf, stab,
             tsem, xsem, gsem0, gsem1, osem0, osem1):
    cid = lax.axis_index("c")
    sid = lax.axis_index("s")
    wid = sid * NC + cid
    gsem = (gsem0, gsem1)
    osem = (osem0, osem1)

    tcp = pltpu.async_copy(tab_hbm.at[pl.ds(sid * RPS, RPS)],
                           stab.at[pl.ds(sid * RPS, RPS)], tsem)
    pltpu.sync_copy(xlen_hbm, lenbuf)
    bs = []
    cof = []
    xcps = []
    for k in range(NIT):
        bs.append(2 * k)
        bk = 2 * k + lax.rem(wid, 2)
        ck = lax.rem(wid // 2 + STRIDE * k, NCH)
        bs[k] = bk
        cof.append(ck * C)
        xcps.append([pltpu.async_copy(x_hbm.at[bk, q, pl.ds(cof[k], C)],
                                      xbuf.at[k, q], xsem) for q in range(Q)])

    iota = lax.iota(jnp.int32, L)
    lv = lenbuf[...]

    def zero_acc(p):
        def body(r, _):
            for v in range(VPR):
                acc[p, r, pl.ds(v * L, L)] = jnp.zeros((L,), jnp.float32)
            return 0
        lax.fori_loop(0, C, body, 0)

    zero_acc(0)
    pltpu.sync_copy(acc.at[0, pl.ds(0, ZPS)],
                    stab.at[pl.ds(ZROW + sid * ZPS, ZPS)])

    active = []
    for k in range(NIT):
        for cp in xcps[k]:
            cp.wait()
        par = lax.rem(wid, 2)
        slen = lax.select(par == 1, lv[2 * k + 1], lv[2 * k])
        lenk = jnp.full((L,), slen, jnp.int32)
        active.append(cof[k] < slen)

        def idx_body(j, _, k=k, lenk=lenk):
            tv = (cof[k] + j * L) + iota
            m = tv < lenk
            for q in range(Q):
                xv = xbuf[k, q, pl.ds(j * L, L)]
                idxbuf[k, q, pl.ds(j * L, L)] = jnp.where(m, xv + q * K, ZROW)
            return 0

        lax.fori_loop(0, C // L, idx_body, 0)

    tcp.wait()
    plsc.subcore_barrier()

    def fire_gathers(k, p):
        @pl.when(active[k])
        def _():
            for q in range(Q):
                pltpu.async_copy(stab.at[idxbuf.at[k, q]], acc.at[p],
                                 gsem[p], add=True)

    def drain_gathers(k, p):
        @pl.when(active[k])
        def _():
            for q in range(Q):
                pltpu.make_async_copy(stab.at[idxbuf.at[k, q]], acc.at[p],
                                      gsem[p]).wait()

    def fire_out(k, p):
        pltpu.async_copy(acc.at[p], out_hbm.at[bs[k], pl.ds(cof[k], C)],
                         osem[p])

    def drain_out(k, p):
        pltpu.make_async_copy(acc.at[p], out_hbm.at[bs[k], pl.ds(cof[k], C)],
                              osem[p]).wait()

    fire_gathers(0, 0)
    zero_acc(1)
    for k in range(1, NIT):
        p, pp = k % 2, (k - 1) % 2
        if k >= 2:
            drain_out(k - 2, p)
            zero_acc(p)
        fire_gathers(k, p)
        drain_gathers(k - 1, pp)
        fire_out(k - 1, pp)
    drain_gathers(NIT - 1, (NIT - 1) % 2)
    fire_out(NIT - 1, (NIT - 1) % 2)
    drain_out(NIT - 2, NIT % 2)
    drain_out(NIT - 1, (NIT - 1) % 2)


def kernel(x, x_len, codebooks):
    return _dequant(x, x_len, codebooks.reshape(Q * K, D))


# speedup vs baseline: 1.0923x; 1.0923x over previous
import functools

import jax
import jax.numpy as jnp
from jax import lax
from jax.experimental import pallas as pl
from jax.experimental.pallas import tpu as pltpu
from jax.experimental.pallas import tpu_sc as plsc

B, Q, T, K, D = 16, 8, 2048, 1024, 128
NC, NS, L = 2, 16, 16
NW = NC * NS
C = 64
NCH = T // C
NIT = B * NCH // NW
STRIDE = 5
ZROW = Q * K
VPR = D // L
NZ = 64
NTAB = Q * K + NZ
RPS = (Q * K) // NS
ZPS = NZ // NS
NB = 2

_mesh = plsc.VectorSubcoreMesh(core_axis_name="c", subcore_axis_name="s")


@functools.partial(
    pl.kernel,
    out_type=jax.ShapeDtypeStruct((B, T, D), jnp.float32),
    mesh=_mesh,
    scratch_types=[
        pltpu.VMEM((NIT, Q, C), jnp.int32),
        pltpu.VMEM((NIT, Q, C), jnp.int32),
        pltpu.VMEM((NB, C, D), jnp.float32),
        pltpu.VMEM((C, D), jnp.float32),
        pltpu.VMEM((L,), jnp.int32),
        pltpu.VMEM_SHARED((NTAB, D), jnp.float32),
        pltpu.SemaphoreType.DMA,
        pltpu.SemaphoreType.DMA,
        pltpu.SemaphoreType.DMA,
        pltpu.SemaphoreType.DMA,
        pltpu.SemaphoreType.DMA,
        pltpu.SemaphoreType.DMA,
        pltpu.SemaphoreType.DMA,
        pltpu.SemaphoreType.DMA,
        pltpu.SemaphoreType.DMA,
    ],
)
def _dequant(x_hbm, xlen_hbm, tab_hbm, out_hbm,
             xbuf, idxbuf, acc, zbuf, lenbuf, stab,
             tsem, xsem, zosem, gsem0, gsem1, gsem2, osem0, osem1, osem2):
    cid = lax.axis_index("c")
    sid = lax.axis_index("s")
    wid = sid * NC + cid
    gsem = (gsem0, gsem1, gsem2)
    osem = (osem0, osem1, osem2)

    tcp = pltpu.async_copy(tab_hbm.at[pl.ds(sid * RPS, RPS)],
                           stab.at[pl.ds(sid * RPS, RPS)], tsem)
    pltpu.sync_copy(xlen_hbm, lenbuf)
    lv = lenbuf[...]
    iota = lax.iota(jnp.int32, L)

    cof = []
    active = []
    xcps = []
    for k in range(NIT):
        ck = lax.rem(wid + STRIDE * k, NCH)
        cof.append(ck * C)
        active.append(cof[k] < lv[k])

        @pl.when(active[k])
        def _(k=k):
            for q in range(Q):
                pltpu.async_copy(x_hbm.at[k, q, pl.ds(cof[k], C)],
                                 xbuf.at[k, q], xsem)

    def zero_block(ref):
        def body(r, _):
            for v in range(VPR):
                ref[r, pl.ds(v * L, L)] = jnp.zeros((L,), jnp.float32)
            return 0
        lax.fori_loop(0, C, body, 0)

    zero_block(zbuf)
    pltpu.sync_copy(zbuf.at[pl.ds(0, ZPS)],
                    stab.at[pl.ds(ZROW + sid * ZPS, ZPS)])

    for k in range(NIT):
        @pl.when(jnp.logical_not(active[k]))
        def _(k=k):
            pltpu.async_copy(zbuf, out_hbm.at[k, pl.ds(cof[k], C)], zosem)

    for k in range(NIT):
        @pl.when(active[k])
        def _(k=k):
            for q in range(Q):
                pltpu.make_async_copy(x_hbm.at[k, q, pl.ds(cof[k], C)],
                                      xbuf.at[k, q], xsem).wait()
            lenk = jnp.full((L,), lv[k], jnp.int32)

            def idx_body(j, _):
                tv = (cof[k] + j * L) + iota
                m = tv < lenk
                for q in range(Q):
                    xv = xbuf[k, q, pl.ds(j * L, L)]
                    idxbuf[k, q, pl.ds(j * L, L)] = jnp.where(
                        m, xv + q * K, ZROW)
                return 0

            lax.fori_loop(0, C // L, idx_body, 0)

    tcp.wait()
    plsc.subcore_barrier()

    def zero_acc(k, p):
        @pl.when(active[k])
        def _():
            def body(r, _):
                for v in range(VPR):
                    acc[p, r, pl.ds(v * L, L)] = jnp.zeros((L,), jnp.float32)
                return 0
            lax.fori_loop(0, C, body, 0)

    def fire_gathers(k, p):
        @pl.when(active[k])
        def _():
            for q in range(Q):
                pltpu.async_copy(stab.at[idxbuf.at[k, q]], acc.at[p],
                                 gsem[p], add=True)

    def drain_gathers(k, p):
        @pl.when(active[k])
        def _():
            for q in range(Q):
                pltpu.make_async_copy(stab.at[idxbuf.at[k, q]], acc.at[p],
                                      gsem[p]).wait()

    def fire_out(k, p):
        @pl.when(active[k])
        def _():
            pltpu.async_copy(acc.at[p], out_hbm.at[k, pl.ds(cof[k], C)],
                             osem[p])

    def drain_out(k, p):
        @pl.when(active[k])
        def _():
            pltpu.make_async_copy(acc.at[p],
                                  out_hbm.at[k, pl.ds(cof[k], C)],
                                  osem[p]).wait()

    for k in range(NIT + NB - 1):
        p = k % NB
        if k < NIT:
            if k >= NB:
                drain_out(k - NB, p)
            zero_acc(k, p)
            fire_gathers(k, p)
        kr = k - (NB - 1)
        if 0 <= kr < NIT:
            pr = kr % NB
            drain_gathers(kr, pr)
            fire_out(kr, pr)
    for k in range(NIT - NB, NIT):
        if k >= 0:
            drain_out(k, k % NB)

    for k in range(NIT):
        @pl.when(jnp.logical_not(active[k]))
        def _(k=k):
            pltpu.make_async_copy(zbuf, out_hbm.at[k, pl.ds(cof[k], C)],
                                  zosem).wait()


def kernel(x, x_len, codebooks):
    return _dequant(x, x_len, codebooks.reshape(Q * K, D))


# speedup vs baseline: 1.0938x; 1.0014x over previous
import functools

import jax
import jax.numpy as jnp
from jax import lax
from jax.experimental import pallas as pl
from jax.experimental.pallas import tpu as pltpu
from jax.experimental.pallas import tpu_sc as plsc

B, Q, T, K, D = 16, 8, 2048, 1024, 128
NC, NS, L = 2, 16, 16
NW = NC * NS
C = 64
NCH = T // C
NIT = B * NCH // NW
STRIDE = 5
ZROW = Q * K
VPR = D // L
NZ = 64
NTAB = Q * K + NZ
RPS = (Q * K) // NS
ZPS = NZ // NS
NB = 2

_mesh = plsc.VectorSubcoreMesh(core_axis_name="c", subcore_axis_name="s")


@functools.partial(
    pl.kernel,
    out_type=jax.ShapeDtypeStruct((B, T, D), jnp.float32),
    mesh=_mesh,
    scratch_types=[
        pltpu.VMEM_SHARED((NTAB, D), jnp.float32),
        pltpu.VMEM((NIT, Q, C), jnp.int32),
        pltpu.VMEM((NIT, Q, C), jnp.int32),
        pltpu.VMEM((NB, C, D), jnp.float32),
        pltpu.VMEM((C, D), jnp.float32),
        pltpu.VMEM((L,), jnp.int32),
        pltpu.SemaphoreType.DMA,
        pltpu.SemaphoreType.DMA,
        pltpu.SemaphoreType.DMA,
        pltpu.SemaphoreType.DMA,
        pltpu.SemaphoreType.DMA,
        pltpu.SemaphoreType.DMA,
    ],
)
def _dequant(x_hbm, xlen_hbm, tab_hbm, out_hbm,
             stab, xbuf, idxbuf, acc, zbuf, lenbuf,
             tsem, xsem, gsem0, gsem1, osem0, osem1):
    zosem = xsem
    cid = lax.axis_index("c")
    sid = lax.axis_index("s")
    wid = sid * NC + cid
    gsem = (gsem0, gsem1)
    osem = (osem0, osem1)

    tcp = pltpu.async_copy(tab_hbm.at[pl.ds(sid * RPS, RPS)],
                           stab.at[pl.ds(sid * RPS, RPS)], tsem)
    pltpu.sync_copy(xlen_hbm, lenbuf)
    lv = lenbuf[...]
    iota = lax.iota(jnp.int32, L)

    cof = []
    active = []
    xcps = []
    for k in range(NIT):
        ck = lax.rem(wid + STRIDE * k, NCH)
        cof.append(ck * C)
        active.append(cof[k] < lv[k])

        @pl.when(active[k])
        def _(k=k):
            for q in range(Q):
                pltpu.async_copy(x_hbm.at[k, q, pl.ds(cof[k], C)],
                                 xbuf.at[k, q], xsem)

    def zero_block(ref):
        def body(r, _):
            for v in range(VPR):
                ref[r, pl.ds(v * L, L)] = jnp.zeros((L,), jnp.float32)
            return 0
        lax.fori_loop(0, C, body, 0)

    zero_block(zbuf)
    pltpu.sync_copy(zbuf.at[pl.ds(0, ZPS)],
                    stab.at[pl.ds(ZROW + sid * ZPS, ZPS)])

    for k in range(NIT):
        @pl.when(active[k])
        def _(k=k):
            for q in range(Q):
                pltpu.make_async_copy(x_hbm.at[k, q, pl.ds(cof[k], C)],
                                      xbuf.at[k, q], xsem).wait()
            lenk = jnp.full((L,), lv[k], jnp.int32)

            def idx_body(j, _):
                tv = (cof[k] + j * L) + iota
                m = tv < lenk
                for q in range(Q):
                    xv = xbuf[k, q, pl.ds(j * L, L)]
                    idxbuf[k, q, pl.ds(j * L, L)] = jnp.where(
                        m, xv + q * K, ZROW)
                return 0

            lax.fori_loop(0, C // L, idx_body, 0)

    for k in range(NIT):
        @pl.when(jnp.logical_not(active[k]))
        def _(k=k):
            pltpu.async_copy(zbuf, out_hbm.at[k, pl.ds(cof[k], C)], zosem)

    tcp.wait()
    plsc.subcore_barrier()

    def zero_acc(k, p):
        @pl.when(active[k])
        def _():
            def body(r, _):
                for v in range(VPR):
                    acc[p, r, pl.ds(v * L, L)] = jnp.zeros((L,), jnp.float32)
                return 0
            lax.fori_loop(0, C, body, 0)

    def fire_gathers(k, p):
        @pl.when(active[k])
        def _():
            for q in range(Q):
                pltpu.async_copy(stab.at[idxbuf.at[k, q]], acc.at[p],
                                 gsem[p], add=True)

    def drain_gathers(k, p):
        @pl.when(active[k])
        def _():
            for q in range(Q):
                pltpu.make_async_copy(stab.at[idxbuf.at[k, q]], acc.at[p],
                                      gsem[p]).wait()

    def fire_out(k, p):
        @pl.when(active[k])
        def _():
            pltpu.async_copy(acc.at[p], out_hbm.at[k, pl.ds(cof[k], C)],
                             osem[p])

    def drain_out(k, p):
        @pl.when(active[k])
        def _():
            pltpu.make_async_copy(acc.at[p],
                                  out_hbm.at[k, pl.ds(cof[k], C)],
                                  osem[p]).wait()

    for k in range(NIT + NB - 1):
        p = k % NB
        if k < NIT:
            if k >= NB:
                drain_out(k - NB, p)
            zero_acc(k, p)
            fire_gathers(k, p)
        kr = k - (NB - 1)
        if 0 <= kr < NIT:
            pr = kr % NB
            drain_gathers(kr, pr)
            fire_out(kr, pr)
    for k in range(NIT - NB, NIT):
        if k >= 0:
            drain_out(k, k % NB)

    for k in range(NIT):
        @pl.when(jnp.logical_not(active[k]))
        def _(k=k):
            pltpu.make_async_copy(zbuf, out_hbm.at[k, pl.ds(cof[k], C)],
                                  zosem).wait()


def kernel(x, x_len, codebooks):
    return _dequant(x, x_len, codebooks.reshape(Q * K, D))
